# baseline (speedup 1.0000x reference)
import functools

import jax
import jax.numpy as jnp
from jax import lax
from jax.experimental import pallas as pl
from jax.experimental.pallas import tpu as pltpu
from jax.experimental.pallas import tpu_sc as plsc

N1 = 4096
N2 = 4096
RC_BITS = 12
L = 16
NC = 2
NS = 16
NW = NC * NS
D0 = 8
BIGC = 1 << 30
SENT_RC = (1 << 24) - 1
BREAK_RC = -1
NEGV = -1.0

_TAKE_DNUMS = lax.GatherDimensionNumbers(
    offset_dims=(), collapsed_slice_dims=(0,), start_index_map=(0,))


def _take(x, idx):
  return lax.gather(x, idx[:, None], _TAKE_DNUMS, slice_sizes=(1,),
                    mode=lax.GatherScatterMode.PROMISE_IN_BOUNDS)


def _lex_improves(v_new, rc_new, v_old, rc_old):
  return (v_new > v_old) | ((v_new == v_old) & (rc_new < rc_old))


def _row(rc):
  return lax.shift_right_logical(rc, RC_BITS)


def _sc_partial_body(nvecs, nnz, vals_hbm, rc_hbm, pv_hbm, pc_hbm,
                     vals_v, rc_v, fv, frc, accv, accrc, sem1, sem2):
  wid = lax.axis_index("s") * NC + lax.axis_index("c")
  ch = nvecs * L
  base_a = (nnz - ch) & ~7
  tail_at = base_a + ch
  rem = nnz - tail_at
  is_last = wid == NW - 1
  base = jnp.minimum(wid * ch, base_a)

  vals_v[pl.ds(0, L)] = jnp.full((L,), NEGV, jnp.float32)
  rc_v[pl.ds(0, L)] = jnp.full((L,), SENT_RC, jnp.int32)
  fv[pl.ds(0, L)] = jnp.full((L,), NEGV, jnp.float32)
  frc[pl.ds(0, L)] = jnp.full((L,), SENT_RC, jnp.int32)
  vals_v[pl.ds(D0 + ch, L)] = jnp.full((L,), NEGV, jnp.float32)
  rc_v[pl.ds(D0 + ch, L)] = jnp.full((L,), BREAK_RC, jnp.int32)

  h1 = pltpu.async_copy(vals_hbm.at[pl.ds(base, ch)],
                        vals_v.at[pl.ds(D0, ch)], sem1)
  h2 = pltpu.async_copy(rc_hbm.at[pl.ds(base, ch)],
                        rc_v.at[pl.ds(D0, ch)], sem2)

  if rem:
    @pl.when(is_last)
    def _tail():
      rc_v[pl.ds(D0 + ch, L)] = jnp.full((L,), SENT_RC, jnp.int32)
      vals_v[pl.ds(D0 + ch + L, L)] = jnp.full((L,), NEGV, jnp.float32)
      rc_v[pl.ds(D0 + ch + L, L)] = jnp.full((L,), BREAK_RC, jnp.int32)
      pltpu.sync_copy(vals_hbm.at[pl.ds(tail_at, rem)],
                      vals_v.at[pl.ds(D0 + ch, rem)])
      pltpu.sync_copy(rc_hbm.at[pl.ds(tail_at, rem)],
                      rc_v.at[pl.ds(D0 + ch, rem)])

  def init(j, _):
    accv[pl.ds(j * L, L)] = jnp.full((L,), NEGV, jnp.float32)
    accrc[pl.ds(j * L, L)] = jnp.full((L,), BIGC, jnp.int32)
    return 0

  lax.fori_loop(0, N1 // L, init, 0)
  h1.wait()
  h2.wait()

  iota = lax.iota(jnp.int32, L)
  idxs = [jnp.maximum(iota - d, 0) for d in (4, 8)]

  def scan_vreg(src_v, src_rc, o):
    v = src_v[pl.ds(o, L)]
    rc = src_rc[pl.ds(o, L)]
    r = _row(rc)
    v1 = src_v[pl.ds(o - 1, L)]
    rc1 = src_rc[pl.ds(o - 1, L)]
    v2 = src_v[pl.ds(o - 2, L)]
    rc2 = src_rc[pl.ds(o - 2, L)]
    v3 = src_v[pl.ds(o - 3, L)]
    rc3 = src_rc[pl.ds(o - 3, L)]
    r2 = _row(rc2)
    tk = (_row(rc1) == r) & _lex_improves(v1, rc1, v, rc)
    v = jnp.where(tk, v1, v)
    rc = jnp.where(tk, rc1, rc)
    tk = (_row(rc3) == r2) & _lex_improves(v3, rc3, v2, rc2)
    v2 = jnp.where(tk, v3, v2)
    rc2 = jnp.where(tk, rc3, rc2)
    tk = (r2 == r) & _lex_improves(v2, rc2, v, rc)
    v = jnp.where(tk, v2, v)
    rc = jnp.where(tk, rc2, rc)
    for idx in idxs:
      vd = _take(v, idx)
      rcd = _take(rc, idx)
      tk = (_row(rcd) == r) & _lex_improves(vd, rcd, v, rc)
      v = jnp.where(tk, vd, v)
      rc = jnp.where(tk, rcd, rc)
    last = (_row(src_rc[pl.ds(o + 1, L)]) != r) | (iota == L - 1)
    return v, rc, r, last

  def append_vreg(i, cnt):
    v, rc, _, last = scan_vreg(vals_v, rc_v, D0 + i * L)
    plsc.store_compressed(fv.at[pl.ds(cnt, L)], v, mask=last)
    plsc.store_compressed(frc.at[pl.ds(cnt, L)], rc, mask=last)
    return cnt + jnp.sum(last.astype(jnp.int32))

  def step(i, cnt):
    cnt = append_vreg(2 * i, cnt)
    return append_vreg(2 * i + 1, cnt)

  cnt = lax.fori_loop(0, nvecs // 2, step, jnp.int32(D0))
  for j in range(nvecs - nvecs // 2 * 2):
    cnt = append_vreg(nvecs // 2 * 2 + j, cnt)
  if rem:
    cnt = lax.cond(is_last, lambda: append_vreg(nvecs, cnt), lambda: cnt)

  fv[pl.ds(cnt, L)] = jnp.full((L,), NEGV, jnp.float32)
  frc[pl.ds(cnt, L)] = jnp.full((L,), SENT_RC, jnp.int32)
  fv[pl.ds(cnt + L, L)] = jnp.full((L,), NEGV, jnp.float32)
  frc[pl.ds(cnt + L, L)] = jnp.full((L,), SENT_RC, jnp.int32)

  def merge_vreg(i, _):
    v, rc, r, last = scan_vreg(fv, frc, D0 + i * L)
    av = plsc.load_gather(accv, [r], mask=last)
    arc = plsc.load_gather(accrc, [r], mask=last)
    upd = last & _lex_improves(v, rc, av, arc)
    plsc.store_scatter(accv, [r], v, mask=upd)
    plsc.store_scatter(accrc, [r], rc, mask=upd)
    return 0

  lax.fori_loop(0, (cnt - D0 + L - 1) // L, merge_vreg, 0)

  pltpu.sync_copy(accv, pv_hbm.at[wid])
  pltpu.sync_copy(accrc, pc_hbm.at[wid])


def _tc_combine_body(pv_ref, prc_ref, outv_ref, outc_ref):
  bv = pv_ref[0, :]
  brc = prc_ref[0, :]
  for w in range(1, NW):
    xv = pv_ref[w, :]
    xrc = prc_ref[w, :]
    upd = _lex_improves(xv, xrc, bv, brc)
    bv = jnp.where(upd, xv, bv)
    brc = jnp.where(upd, xrc, brc)
  outv_ref[:] = jnp.maximum(bv, 0.0)
  outc_ref[:] = jnp.where(bv > 0.0, brc & (N2 - 1), 0)


@jax.jit
def kernel(sim_values, sim_rows, sim_cols):
  nnz = sim_values.shape[0]
  nvecs = -(-nnz // (NW * L))
  rc = lax.shift_left(sim_rows.astype(jnp.int32), RC_BITS) | (
      sim_cols.astype(jnp.int32))

  mesh = plsc.VectorSubcoreMesh(
      core_axis_name="c", subcore_axis_name="s", num_cores=NC,
      num_subcores=NS)

  ch = nvecs * L
  partial = pl.kernel(
      functools.partial(_sc_partial_body, nvecs, nnz),
      compiler_params=pltpu.CompilerParams(needs_layout_passes=False),
      out_type=(
          jax.ShapeDtypeStruct((NW, N1), jnp.float32),
          jax.ShapeDtypeStruct((NW, N1), jnp.int32),
      ),
      mesh=mesh,
      scratch_types=[
          pltpu.VMEM((D0 + ch + 2 * L,), jnp.float32),
          pltpu.VMEM((D0 + ch + 2 * L,), jnp.int32),
          pltpu.VMEM((D0 + ch + 2 * L,), jnp.float32),
          pltpu.VMEM((D0 + ch + 2 * L,), jnp.int32),
          pltpu.VMEM((N1,), jnp.float32),
          pltpu.VMEM((N1,), jnp.int32),
          pltpu.SemaphoreType.DMA,
          pltpu.SemaphoreType.DMA,
      ],
  )
  pv, prc = partial(sim_values, rc)

  scores, indices = pl.pallas_call(
      _tc_combine_body,
      out_shape=(
          jax.ShapeDtypeStruct((N1,), jnp.float32),
          jax.ShapeDtypeStruct((N1,), jnp.int32),
      ),
  )(pv, prc)
  return scores, indices


# speedup vs baseline: 1.0973x; 1.0973x over previous
import functools

import jax
import jax.numpy as jnp
from jax import lax
from jax.experimental import pallas as pl
from jax.experimental.pallas import tpu as pltpu
from jax.experimental.pallas import tpu_sc as plsc

N1 = 4096
N2 = 4096
RC_BITS = 12
L = 16
NC = 2
NS = 16
NW = NC * NS
D0 = 8
BIGC = 1 << 30
SENT_RC = (1 << 24) - 1
BREAK_RC = -1
NEGV = -1.0

_TAKE_DNUMS = lax.GatherDimensionNumbers(
    offset_dims=(), collapsed_slice_dims=(0,), start_index_map=(0,))


def _take(x, idx):
  return lax.gather(x, idx[:, None], _TAKE_DNUMS, slice_sizes=(1,),
                    mode=lax.GatherScatterMode.PROMISE_IN_BOUNDS)


def _lex_improves(v_new, rc_new, v_old, rc_old):
  return (v_new > v_old) | ((v_new == v_old) & (rc_new < rc_old))


def _row(rc):
  return lax.shift_right_logical(rc, RC_BITS)


def _sc_partial_body(nvecs, nnz, vals_hbm, rc_hbm, pv_hbm, pc_hbm,
                     vals_v, rc_v, fv, frc, accv, accrc, sem1, sem2):
  wid = lax.axis_index("s") * NC + lax.axis_index("c")
  ch = nvecs * L
  base_a = (nnz - ch) & ~7
  tail_at = base_a + ch
  rem = nnz - tail_at
  is_last = wid == NW - 1
  base = jnp.minimum(wid * ch, base_a)

  vals_v[pl.ds(0, L)] = jnp.full((L,), NEGV, jnp.float32)
  rc_v[pl.ds(0, L)] = jnp.full((L,), SENT_RC, jnp.int32)
  fv[pl.ds(0, L)] = jnp.full((L,), NEGV, jnp.float32)
  frc[pl.ds(0, L)] = jnp.full((L,), SENT_RC, jnp.int32)
  vals_v[pl.ds(D0 + ch, L)] = jnp.full((L,), NEGV, jnp.float32)
  rc_v[pl.ds(D0 + ch, L)] = jnp.full((L,), BREAK_RC, jnp.int32)

  h1 = pltpu.async_copy(vals_hbm.at[pl.ds(base, ch)],
                        vals_v.at[pl.ds(D0, ch)], sem1)
  h2 = pltpu.async_copy(rc_hbm.at[pl.ds(base, ch)],
                        rc_v.at[pl.ds(D0, ch)], sem2)

  if rem:
    @pl.when(is_last)
    def _tail():
      rc_v[pl.ds(D0 + ch, L)] = jnp.full((L,), SENT_RC, jnp.int32)
      vals_v[pl.ds(D0 + ch + L, L)] = jnp.full((L,), NEGV, jnp.float32)
      rc_v[pl.ds(D0 + ch + L, L)] = jnp.full((L,), BREAK_RC, jnp.int32)
      pltpu.sync_copy(vals_hbm.at[pl.ds(tail_at, rem)],
                      vals_v.at[pl.ds(D0 + ch, rem)])
      pltpu.sync_copy(rc_hbm.at[pl.ds(tail_at, rem)],
                      rc_v.at[pl.ds(D0 + ch, rem)])

  def init(j, _):
    accv[pl.ds(j * L, L)] = jnp.full((L,), NEGV, jnp.float32)
    accrc[pl.ds(j * L, L)] = jnp.full((L,), BIGC, jnp.int32)
    return 0

  lax.fori_loop(0, N1 // L, init, 0)
  h1.wait()
  h2.wait()

  iota = lax.iota(jnp.int32, L)
  idxs = [jnp.maximum(iota - d, 0) for d in (4, 8)]

  def scan_vreg(src_v, src_rc, o):
    v = src_v[pl.ds(o, L)]
    rc = src_rc[pl.ds(o, L)]
    r = _row(rc)
    v1 = src_v[pl.ds(o - 1, L)]
    rc1 = src_rc[pl.ds(o - 1, L)]
    v2 = src_v[pl.ds(o - 2, L)]
    rc2 = src_rc[pl.ds(o - 2, L)]
    v3 = src_v[pl.ds(o - 3, L)]
    rc3 = src_rc[pl.ds(o - 3, L)]
    r2 = _row(rc2)
    tk = (_row(rc1) == r) & _lex_improves(v1, rc1, v, rc)
    v = jnp.where(tk, v1, v)
    rc = jnp.where(tk, rc1, rc)
    tk = (_row(rc3) == r2) & _lex_improves(v3, rc3, v2, rc2)
    v2 = jnp.where(tk, v3, v2)
    rc2 = jnp.where(tk, rc3, rc2)
    tk = (r2 == r) & _lex_improves(v2, rc2, v, rc)
    v = jnp.where(tk, v2, v)
    rc = jnp.where(tk, rc2, rc)
    for idx in idxs:
      vd = _take(v, idx)
      rcd = _take(rc, idx)
      tk = (_row(rcd) == r) & _lex_improves(vd, rcd, v, rc)
      v = jnp.where(tk, vd, v)
      rc = jnp.where(tk, rcd, rc)
    last = (_row(src_rc[pl.ds(o + 1, L)]) != r) | (iota == L - 1)
    return v, rc, r, last

  def append_vreg(i, cnt):
    v, rc, _, last = scan_vreg(vals_v, rc_v, D0 + i * L)
    plsc.store_compressed(fv.at[pl.ds(cnt, L)], v, mask=last)
    plsc.store_compressed(frc.at[pl.ds(cnt, L)], rc, mask=last)
    return cnt + jnp.sum(last.astype(jnp.int32))

  cnt = plsc.parallel_loop(
      0, nvecs, carry=jnp.int32(D0), unroll=4)(append_vreg)
  if rem:
    cnt = lax.cond(is_last, lambda: append_vreg(nvecs, cnt), lambda: cnt)

  fv[pl.ds(cnt, L)] = jnp.full((L,), NEGV, jnp.float32)
  frc[pl.ds(cnt, L)] = jnp.full((L,), SENT_RC, jnp.int32)
  fv[pl.ds(cnt + L, L)] = jnp.full((L,), NEGV, jnp.float32)
  frc[pl.ds(cnt + L, L)] = jnp.full((L,), SENT_RC, jnp.int32)

  def merge_vreg(i, _):
    v, rc, r, last = scan_vreg(fv, frc, D0 + i * L)
    av = plsc.load_gather(accv, [r], mask=last)
    arc = plsc.load_gather(accrc, [r], mask=last)
    upd = last & _lex_improves(v, rc, av, arc)
    plsc.store_scatter(accv, [r], v, mask=upd)
    plsc.store_scatter(accrc, [r], rc, mask=upd)
    return 0

  lax.fori_loop(0, (cnt - D0 + L - 1) // L, merge_vreg, 0)

  pltpu.sync_copy(accv, pv_hbm.at[wid])
  pltpu.sync_copy(accrc, pc_hbm.at[wid])


def _tc_combine_body(pv_ref, prc_ref, outv_ref, outc_ref):
  bv = pv_ref[0, :]
  brc = prc_ref[0, :]
  for w in range(1, NW):
    xv = pv_ref[w, :]
    xrc = prc_ref[w, :]
    upd = _lex_improves(xv, xrc, bv, brc)
    bv = jnp.where(upd, xv, bv)
    brc = jnp.where(upd, xrc, brc)
  outv_ref[:] = jnp.maximum(bv, 0.0)
  outc_ref[:] = jnp.where(bv > 0.0, brc & (N2 - 1), 0)


@jax.jit
def kernel(sim_values, sim_rows, sim_cols):
  nnz = sim_values.shape[0]
  nvecs = -(-nnz // (NW * L))
  rc = lax.shift_left(sim_rows.astype(jnp.int32), RC_BITS) | (
      sim_cols.astype(jnp.int32))

  mesh = plsc.VectorSubcoreMesh(
      core_axis_name="c", subcore_axis_name="s", num_cores=NC,
      num_subcores=NS)

  ch = nvecs * L
  partial = pl.kernel(
      functools.partial(_sc_partial_body, nvecs, nnz),
      compiler_params=pltpu.CompilerParams(needs_layout_passes=False),
      out_type=(
          jax.ShapeDtypeStruct((NW, N1), jnp.float32),
          jax.ShapeDtypeStruct((NW, N1), jnp.int32),
      ),
      mesh=mesh,
      scratch_types=[
          pltpu.VMEM((D0 + ch + 2 * L,), jnp.float32),
          pltpu.VMEM((D0 + ch + 2 * L,), jnp.int32),
          pltpu.VMEM((D0 + ch + 2 * L,), jnp.float32),
          pltpu.VMEM((D0 + ch + 2 * L,), jnp.int32),
          pltpu.VMEM((N1,), jnp.float32),
          pltpu.VMEM((N1,), jnp.int32),
          pltpu.SemaphoreType.DMA,
          pltpu.SemaphoreType.DMA,
      ],
  )
  pv, prc = partial(sim_values, rc)

  scores, indices = pl.pallas_call(
      _tc_combine_body,
      out_shape=(
          jax.ShapeDtypeStruct((N1,), jnp.float32),
          jax.ShapeDtypeStruct((N1,), jnp.int32),
      ),
  )(pv, prc)
  return scores, indices
